# Initial kernel scaffold; baseline (speedup 1.0000x reference)
import jax
import jax.numpy as jnp
from jax.experimental import pallas as pl


def kernel(x, edge_src, edge_dst, edge_sh, edge_rbf, edge_len, norm_w, norm_b, mlp_w1, mlp_b1, mlp_w2, mlp_b2, mlp_w3, mlp_b3, gate_w1, gate_b1, gate_w2, gate_b2, msg_ws, msg_wg, msg_wv, upd_w0, upd_w1, self_w0, self_w1, res_scale):
    raise NotImplementedError("write your pallas kernel here")


# speedup vs baseline: 2.3223x; 2.3223x over previous
import functools

import numpy as np
import jax
import jax.numpy as jnp
from jax import lax
from jax.experimental import pallas as pl
from jax.experimental.pallas import tpu as pltpu
from jax.experimental.pallas import tpu_sc as plsc

N = 10000
E = 160000
MUL = 16
D = 64
CUTOFF = 1.0
EPS = 1e-8

NC = 2
NS = 16
NW = NC * NS
CHUNK = 128
CH_PER_GROUP = 10
GROUPS = 4
CH_W = CH_PER_GROUP * GROUPS
E_W = CH_W * CHUNK
E_PAD = NW * E_W
GROUP_E = CH_PER_GROUP * CHUNK
TOT_CH = NW * CH_W
N_PAD = 10016
STRIPE = N_PAD // NS
AGG_W = 80

EB = 1024
N_EB = E_PAD // EB

_C3 = float(1.0 / np.sqrt(3.0))
_P0 = float(1.0 / np.sqrt(2.0 * MUL))
_P1 = float(np.sqrt(3.0 / (2.0 * MUL)))
_SCL = float(1.0 / np.sqrt(MUL))

_R_np = np.zeros((16, 256), np.float32)
for _u in range(16):
    _R_np[_u, _u * 16:(_u + 1) * 16] = 1.0
_S_np = np.zeros((256, 16), np.float32)
for _u in range(16):
    for _w in range(16):
        _S_np[_u * 16 + _w, _w] = 1.0


def _silu(v):
    return v * jax.nn.sigmoid(v)


def _prep_body(xp_ref, w_ref, b_ref, out_ref):
    xp = xp_ref[...]
    s = xp[:, :MUL]
    mean = jnp.mean(s, axis=1, keepdims=True)
    var = jnp.mean((s - mean) ** 2, axis=1, keepdims=True)
    sn = (s - mean) * lax.rsqrt(var + EPS)
    vx = xp[:, 16:32]
    vy = xp[:, 32:48]
    vz = xp[:, 48:64]
    ninv = lax.rsqrt((vx * vx + vy * vy + vz * vz) * (1.0 / 3.0) + EPS)
    out = jnp.concatenate([sn, vx * ninv, vy * ninv, vz * ninv], axis=1)
    out_ref[...] = out * w_ref[...] + b_ref[...]


def _prep(x_p, w_p, b_p):
    return pl.pallas_call(
        _prep_body,
        out_shape=jax.ShapeDtypeStruct((N, D), jnp.float32),
    )(x_p, w_p, b_p)


def _gather_sc(table, idx2d):
    mesh = plsc.VectorSubcoreMesh(core_axis_name="c", subcore_axis_name="s")

    @functools.partial(
        pl.kernel,
        mesh=mesh,
        out_type=jax.ShapeDtypeStruct((E_PAD, D), jnp.float32),
        compiler_params=pltpu.CompilerParams(use_tc_tiling_on_sc=False),
        scratch_types=[
            pltpu.VMEM((CH_W, CHUNK), jnp.int32),
            pltpu.VMEM((GROUP_E, D), jnp.float32),
            pltpu.SemaphoreType.DMA,
        ],
    )
    def gk(tab_hbm, idx_hbm, out_hbm, idx_v, gbuf, sem):
        c = lax.axis_index("c")
        s = lax.axis_index("s")
        wid = s * NC + c
        pltpu.sync_copy(idx_hbm.at[pl.ds(wid * CH_W, CH_W), :], idx_v)
        ebase = wid * E_W
        for g in range(GROUPS):
            cps = []
            for j in range(CH_PER_GROUP):
                cps.append(pltpu.async_copy(
                    tab_hbm.at[idx_v.at[g * CH_PER_GROUP + j]],
                    gbuf.at[pl.ds(j * CHUNK, CHUNK), :],
                    sem,
                ))
            for cp in cps:
                cp.wait()
            pltpu.sync_copy(
                gbuf, out_hbm.at[pl.ds(ebase + g * GROUP_E, GROUP_E), :])

    return gk(table, idx2d)


def _main_body(rbf_ref, aux_ref, xs_ref, w1_ref, b1_ref, w2_ref, b2_ref,
               w3_ref, b3_ref, gw1_ref, gb1_ref, gw2_ref, gb2_ref,
               r_ref, s_ref, out_ref):
    f32 = jnp.float32
    rbf = rbf_ref[...]
    h = _silu(jnp.dot(rbf, w1_ref[...], preferred_element_type=f32)
              + b1_ref[...])
    h = _silu(jnp.dot(h, w2_ref[...], preferred_element_type=f32)
              + b2_ref[...])
    tpw = jnp.dot(h, w3_ref[...], preferred_element_type=f32) + b3_ref[...]

    xs = xs_ref[...]
    sfe = xs[:, :16]
    vx = xs[:, 16:32]
    vy = xs[:, 32:48]
    vz = xs[:, 48:64]
    aux = aux_ref[...]
    sh0 = aux[:, 0:1]
    shx = aux[:, 1:2]
    shy = aux[:, 2:3]
    shz = aux[:, 3:4]
    elen = aux[:, 4:5]

    a = sfe * sh0
    bsum = vx * shx + vy * shy + vz * shz

    R = r_ref[...]
    S = s_ref[...]
    ar = jnp.dot(a, R, preferred_element_type=f32)
    br = jnp.dot(bsum, R, preferred_element_type=f32)
    sr = jnp.dot(sfe, R, preferred_element_type=f32)
    vxr = jnp.dot(vx, R, preferred_element_type=f32)
    vyr = jnp.dot(vy, R, preferred_element_type=f32)
    vzr = jnp.dot(vz, R, preferred_element_type=f32)

    t0 = tpw[:, 0:256] * ar + _C3 * (tpw[:, 768:1024] * br)
    m0 = _P0 * jnp.dot(t0, S, preferred_element_type=f32)
    t011 = jnp.dot(tpw[:, 256:512] * sr, S, preferred_element_type=f32)
    tx = jnp.dot(tpw[:, 512:768] * vxr, S, preferred_element_type=f32)
    ty = jnp.dot(tpw[:, 512:768] * vyr, S, preferred_element_type=f32)
    tz = jnp.dot(tpw[:, 512:768] * vzr, S, preferred_element_type=f32)
    p1c = _P1 * _C3
    m1x = p1c * (t011 * shx + tx * sh0)
    m1y = p1c * (t011 * shy + ty * sh0)
    m1z = p1c * (t011 * shz + tz * sh0)

    cut = 0.5 * (jnp.cos(jnp.pi * elen) + 1.0) * (elen <= CUTOFF).astype(f32)
    g = _silu(jnp.dot(rbf, gw1_ref[...], preferred_element_type=f32)
              + gb1_ref[...])
    gw = jax.nn.sigmoid(jnp.dot(g, gw2_ref[...], preferred_element_type=f32)
                        + gb2_ref[...])
    ew = cut * gw

    m = jnp.concatenate([m0, m1x, m1y, m1z], axis=1) * ew
    ew16 = jnp.broadcast_to(ew, (m.shape[0], 16))
    out_ref[...] = jnp.concatenate([m, ew16], axis=1)


def _main(rbf_p, aux_p, xs, w1, b1, w2, b2, w3, b3, gw1, gb1, gw2, gb2, Rm, Sm):
    blk = lambda shp: pl.BlockSpec(shp, lambda i: (0, 0))
    ebk = lambda w: pl.BlockSpec((EB, w), lambda i: (i, 0))
    return pl.pallas_call(
        _main_body,
        grid=(N_EB,),
        in_specs=[
            ebk(16), ebk(8), ebk(64),
            blk((16, 64)), blk((1, 64)), blk((64, 64)), blk((1, 64)),
            blk((64, 1024)), blk((1, 1024)),
            blk((16, 64)), blk((1, 64)), blk((64, 1)), blk((1, 1)),
            blk((16, 256)), blk((256, 16)),
        ],
        out_specs=pl.BlockSpec((EB, AGG_W), lambda i: (i, 0)),
        out_shape=jax.ShapeDtypeStruct((E_PAD, AGG_W), jnp.float32),
    )(rbf_p, aux_p, xs, w1, b1, w2, b2, w3, b3, gw1, gb1, gw2, gb2, Rm, Sm)


def _scatter_sc(m_ext, dst2d, zstripe):
    mesh = plsc.VectorSubcoreMesh(core_axis_name="c", subcore_axis_name="s")

    @functools.partial(
        pl.kernel,
        mesh=mesh,
        out_type=jax.ShapeDtypeStruct((NC, N_PAD, AGG_W), jnp.float32),
        compiler_params=pltpu.CompilerParams(use_tc_tiling_on_sc=False),
        scratch_types=[
            pltpu.VMEM((CH_W, CHUNK), jnp.int32),
            pltpu.VMEM((CHUNK, AGG_W), jnp.float32),
            pltpu.VMEM((CHUNK, AGG_W), jnp.float32),
            pltpu.VMEM_SHARED((N_PAD, AGG_W), jnp.float32),
            pltpu.SemaphoreType.DMA,
            pltpu.SemaphoreType.DMA,
        ],
    )
    def sk(m_hbm, idx_hbm, z_hbm, out_hbm, idx_v, mb0, mb1, acc, sm0, sm1):
        c = lax.axis_index("c")
        s = lax.axis_index("s")
        wid = s * NC + c
        pltpu.sync_copy(idx_hbm.at[pl.ds(wid * CH_W, CH_W), :], idx_v)
        pltpu.sync_copy(z_hbm, acc.at[pl.ds(s * STRIPE, STRIPE), :])
        plsc.subcore_barrier()
        ebase = wid * E_W
        bufs = (mb0, mb1)
        sems = (sm0, sm1)
        cps = [None, None]
        cps[0] = pltpu.async_copy(
            m_hbm.at[pl.ds(ebase, CHUNK), :], mb0, sm0)
        for j in range(CH_W):
            p = j % 2
            cps[p].wait()
            if j + 1 < CH_W:
                q = (j + 1) % 2
                cps[q] = pltpu.async_copy(
                    m_hbm.at[pl.ds(ebase + (j + 1) * CHUNK, CHUNK), :],
                    bufs[q], sems[q])
            pltpu.sync_copy(bufs[p], acc.at[idx_v.at[j]], add=True)
        plsc.subcore_barrier()
        pltpu.sync_copy(
            acc.at[pl.ds(s * STRIPE, STRIPE), :],
            out_hbm.at[c, pl.ds(s * STRIPE, STRIPE), :],
        )

    return sk(m_ext, dst2d, zstripe)


def _epi_body(aggc_ref, xp_ref, xn_ref, mws_ref, mwg_ref, mwv_ref,
              uw0_ref, uw1_ref, sw0_ref, sw1_ref, rs_ref, out_ref):
    f32 = jnp.float32
    agg = aggc_ref[0] + aggc_ref[1]
    agg = agg[:N, :]
    den = jnp.maximum(agg[:, 64:65], 1e-8)
    a = agg[:, :64] / den
    a_s = a[:, :16]
    a_vx = a[:, 16:32]
    a_vy = a[:, 32:48]
    a_vz = a[:, 48:64]

    scal = _silu(jnp.dot(a_s, mws_ref[...], preferred_element_type=f32)
                 * _SCL)
    gts = jax.nn.sigmoid(jnp.dot(a_s, mwg_ref[...],
                                 preferred_element_type=f32) * _SCL)
    mwv = mwv_ref[...]
    vex = gts * (jnp.dot(a_vx, mwv, preferred_element_type=f32) * _SCL)
    vey = gts * (jnp.dot(a_vy, mwv, preferred_element_type=f32) * _SCL)
    vez = gts * (jnp.dot(a_vz, mwv, preferred_element_type=f32) * _SCL)

    xn = xn_ref[...]
    sw0 = sw0_ref[...]
    sw1 = sw1_ref[...]
    uw0 = uw0_ref[...]
    uw1 = uw1_ref[...]
    o_s = (jnp.dot(xn[:, :16], sw0, preferred_element_type=f32)
           + jnp.dot(scal, uw0, preferred_element_type=f32)) * _SCL
    o_vx = (jnp.dot(xn[:, 16:32], sw1, preferred_element_type=f32)
            + jnp.dot(vex, uw1, preferred_element_type=f32)) * _SCL
    o_vy = (jnp.dot(xn[:, 32:48], sw1, preferred_element_type=f32)
            + jnp.dot(vey, uw1, preferred_element_type=f32)) * _SCL
    o_vz = (jnp.dot(xn[:, 48:64], sw1, preferred_element_type=f32)
            + jnp.dot(vez, uw1, preferred_element_type=f32)) * _SCL
    out = jnp.concatenate([o_s, o_vx, o_vy, o_vz], axis=1)
    out_ref[...] = xp_ref[...] + rs_ref[0, 0] * out


def _epilogue(aggc, x_p, xn_p, mws, mwg, mwv, uw0, uw1, sw0, sw1, rs):
    return pl.pallas_call(
        _epi_body,
        out_shape=jax.ShapeDtypeStruct((N, D), jnp.float32),
    )(aggc, x_p, xn_p, mws, mwg, mwv, uw0, uw1, sw0, sw1, rs)


def _planarize(arr64):
    v = arr64[:, 16:].reshape(-1, 16, 3).transpose(0, 2, 1).reshape(-1, 48)
    return jnp.concatenate([arr64[:, :16], v], axis=1)


def _unplanarize(arr64):
    v = arr64[:, 16:].reshape(-1, 3, 16).transpose(0, 2, 1).reshape(-1, 48)
    return jnp.concatenate([arr64[:, :16], v], axis=1)


def kernel(x, edge_src, edge_dst, edge_sh, edge_rbf, edge_len, norm_w,
           norm_b, mlp_w1, mlp_b1, mlp_w2, mlp_b2, mlp_w3, mlp_b3, gate_w1,
           gate_b1, gate_w2, gate_b2, msg_ws, msg_wg, msg_wv, upd_w0, upd_w1,
           self_w0, self_w1, res_scale):
    f32 = jnp.float32
    x_p = _planarize(x)
    wv = norm_w[16:].reshape(16, 3).T.reshape(48)
    w_p = jnp.concatenate([norm_w[:16], wv]).reshape(1, D)
    bv = norm_b[16:].reshape(16, 3).T.reshape(48)
    b_p = jnp.concatenate([norm_b[:16], bv]).reshape(1, D)

    pad = E_PAD - E
    src_p = jnp.pad(edge_src.astype(jnp.int32), (0, pad)).reshape(TOT_CH,
                                                                  CHUNK)
    dst_p = jnp.pad(edge_dst.astype(jnp.int32), (0, pad)).reshape(TOT_CH,
                                                                  CHUNK)
    rbf_p = jnp.pad(edge_rbf, ((0, pad), (0, 0)))
    aux = jnp.concatenate(
        [edge_sh, edge_len[:, None], jnp.zeros((E, 3), f32)], axis=1)
    aux_p = jnp.pad(aux, ((0, pad), (0, 0)),
                    constant_values=0.0).at[E:, 4].set(2.0)

    zstripe = jnp.zeros((STRIPE, AGG_W), f32)
    Rm = jnp.asarray(_R_np)
    Sm = jnp.asarray(_S_np)

    xn_p = _prep(x_p, w_p, b_p)
    xs = _gather_sc(xn_p, src_p)
    m_ext = _main(rbf_p, aux_p, xs,
                  mlp_w1, mlp_b1.reshape(1, -1), mlp_w2,
                  mlp_b2.reshape(1, -1), mlp_w3, mlp_b3.reshape(1, -1),
                  gate_w1, gate_b1.reshape(1, -1), gate_w2,
                  gate_b2.reshape(1, -1), Rm, Sm)
    aggc = _scatter_sc(m_ext, dst_p, zstripe)
    out_p = _epilogue(aggc, x_p, xn_p, msg_ws, msg_wg, msg_wv, upd_w0,
                      upd_w1, self_w0, self_w1,
                      res_scale.reshape(1, 1))
    return _unplanarize(out_p)


# speedup vs baseline: 3.1512x; 1.3569x over previous
import functools

import numpy as np
import jax
import jax.numpy as jnp
from jax import lax
from jax.experimental import pallas as pl
from jax.experimental.pallas import tpu as pltpu
from jax.experimental.pallas import tpu_sc as plsc

N = 10000
E = 160000
MUL = 16
D = 64
CUTOFF = 1.0
EPS = 1e-8

NC = 2
NS = 16
NW = NC * NS
CHUNK = 128
CH_PER_GROUP = 10
GROUPS = 4
CH_W = CH_PER_GROUP * GROUPS
E_W = CH_W * CHUNK
E_PAD = NW * E_W
GROUP_E = CH_PER_GROUP * CHUNK
TOT_CH = NW * CH_W
N_PAD = 10016
TRASH = N_PAD - 1
STRIPE = N_PAD // NS
AGG_W = 80

EB = 2000
N_EB = E // EB

CUT_R = 1256
E_CUT = CUT_R * 128

_C3 = float(1.0 / np.sqrt(3.0))
_P0 = float(1.0 / np.sqrt(2.0 * MUL))
_P1C = float(np.sqrt(3.0 / (2.0 * MUL)) / np.sqrt(3.0))
_SCL = float(1.0 / np.sqrt(MUL))

_R4_np = np.zeros((64, 1024), np.float32)
for _g in range(4):
    for _u in range(16):
        _R4_np[_g * 16 + _u, _g * 256 + _u * 16:_g * 256 + (_u + 1) * 16] = 1.0
_S4_np = np.zeros((1024, 64), np.float32)
for _q, _sc in enumerate([_P0, _P1C, _P1C, _P1C]):
    for _u in range(16):
        for _w in range(16):
            _S4_np[_q * 256 + _u * 16 + _w, _q * 16 + _w] = _sc


def _silu(v):
    return v * jax.nn.sigmoid(v)


def _prep_body(xp_ref, w_ref, b_ref, len_ref, out_ref, cut_ref):
    xp = xp_ref[...]
    s = xp[:, :MUL]
    mean = jnp.mean(s, axis=1, keepdims=True)
    var = jnp.mean((s - mean) ** 2, axis=1, keepdims=True)
    sn = (s - mean) * lax.rsqrt(var + EPS)
    vx = xp[:, 16:32]
    vy = xp[:, 32:48]
    vz = xp[:, 48:64]
    ninv = lax.rsqrt((vx * vx + vy * vy + vz * vz) * (1.0 / 3.0) + EPS)
    out = jnp.concatenate([sn, vx * ninv, vy * ninv, vz * ninv], axis=1)
    out_ref[...] = out * w_ref[...] + b_ref[...]
    el = len_ref[...]
    cut_ref[...] = (0.5 * (jnp.cos(jnp.pi * el) + 1.0)
                    * (el <= CUTOFF).astype(jnp.float32))


def _prep(x_p, w_p, b_p, len2d):
    return pl.pallas_call(
        _prep_body,
        out_shape=(jax.ShapeDtypeStruct((N, D), jnp.float32),
                   jax.ShapeDtypeStruct((CUT_R, 128), jnp.float32)),
    )(x_p, w_p, b_p, len2d)


def _gather_sc(table, idx2d):
    mesh = plsc.VectorSubcoreMesh(core_axis_name="c", subcore_axis_name="s")

    @functools.partial(
        pl.kernel,
        mesh=mesh,
        out_type=jax.ShapeDtypeStruct((E_PAD, D), jnp.float32),
        compiler_params=pltpu.CompilerParams(use_tc_tiling_on_sc=False),
        scratch_types=[
            pltpu.VMEM((CH_W, CHUNK), jnp.int32),
            pltpu.VMEM((GROUP_E, D), jnp.float32),
            pltpu.SemaphoreType.DMA,
        ],
    )
    def gk(tab_hbm, idx_hbm, out_hbm, idx_v, gbuf, sem):
        c = lax.axis_index("c")
        s = lax.axis_index("s")
        wid = s * NC + c
        pltpu.sync_copy(idx_hbm.at[pl.ds(wid * CH_W, CH_W), :], idx_v)
        ebase = wid * E_W
        for g in range(GROUPS):
            cps = []
            for j in range(CH_PER_GROUP):
                cps.append(pltpu.async_copy(
                    tab_hbm.at[idx_v.at[g * CH_PER_GROUP + j]],
                    gbuf.at[pl.ds(j * CHUNK, CHUNK), :],
                    sem,
                ))
            for cp in cps:
                cp.wait()
            pltpu.sync_copy(
                gbuf, out_hbm.at[pl.ds(ebase + g * GROUP_E, GROUP_E), :])

    return gk(table, idx2d)


def _main_body(rbf_ref, sh_ref, cut_ref, xs_ref, w1_ref, b1_ref, w2_ref,
               b2_ref, w3_ref, b3_ref, gw1_ref, gb1_ref, gw2_ref, gb2_ref,
               r4_ref, s4_ref, out_ref):
    f32 = jnp.float32
    rbf = rbf_ref[...]
    h = _silu(jnp.dot(rbf, w1_ref[...], preferred_element_type=f32)
              + b1_ref[...])
    h = _silu(jnp.dot(h, w2_ref[...], preferred_element_type=f32)
              + b2_ref[...])
    tpw = jnp.dot(h, w3_ref[...], preferred_element_type=f32) + b3_ref[...]

    xs = xs_ref[...]
    sh = sh_ref[...]
    sh0 = sh[:, 0:1]
    shx = sh[:, 1:2]
    shy = sh[:, 2:3]
    shz = sh[:, 3:4]

    XR = jnp.dot(xs, r4_ref[...], preferred_element_type=f32)
    sr = XR[:, 0:256]
    vxr = XR[:, 256:512]
    vyr = XR[:, 512:768]
    vzr = XR[:, 768:1024]

    ar = sr * sh0
    br = vxr * shx + vyr * shy + vzr * shz
    t0 = tpw[:, 0:256] * ar + _C3 * (tpw[:, 768:1024] * br)
    c011 = tpw[:, 256:512] * sr
    tq = tpw[:, 512:768]
    txp = c011 * shx + (tq * vxr) * sh0
    typ = c011 * shy + (tq * vyr) * sh0
    tzp = c011 * shz + (tq * vzr) * sh0
    T = jnp.concatenate([t0, txp, typ, tzp], axis=1)
    m = jnp.dot(T, s4_ref[...], preferred_element_type=f32)

    g = _silu(jnp.dot(rbf, gw1_ref[...], preferred_element_type=f32)
              + gb1_ref[...])
    gw = jax.nn.sigmoid(jnp.dot(g, gw2_ref[...], preferred_element_type=f32)
                        + gb2_ref[...])
    ew = cut_ref[...] * gw

    m = m * ew
    ew16 = jnp.broadcast_to(ew, (m.shape[0], 16))
    out_ref[...] = jnp.concatenate([m, ew16], axis=1)


def _main(rbf, sh, cut, xs, w1, b1, w2, b2, w3, b3, gw1, gb1, gw2, gb2,
          R4m, S4m):
    blk = lambda shp: pl.BlockSpec(shp, lambda i: (0, 0))
    ebk = lambda w: pl.BlockSpec((EB, w), lambda i: (i, 0))
    return pl.pallas_call(
        _main_body,
        grid=(N_EB,),
        in_specs=[
            ebk(16), ebk(4), ebk(1), ebk(64),
            blk((16, 64)), blk((1, 64)), blk((64, 64)), blk((1, 64)),
            blk((64, 1024)), blk((1, 1024)),
            blk((16, 64)), blk((1, 64)), blk((64, 1)), blk((1, 1)),
            blk((64, 1024)), blk((1024, 64)),
        ],
        out_specs=pl.BlockSpec((EB, AGG_W), lambda i: (i, 0)),
        out_shape=jax.ShapeDtypeStruct((E_PAD, AGG_W), jnp.float32),
    )(rbf, sh, cut, xs, w1, b1, w2, b2, w3, b3, gw1, gb1, gw2, gb2,
      R4m, S4m)


def _scatter_sc(m_ext, dst2d, zstripe):
    mesh = plsc.VectorSubcoreMesh(core_axis_name="c", subcore_axis_name="s")

    @functools.partial(
        pl.kernel,
        mesh=mesh,
        out_type=jax.ShapeDtypeStruct((NC, N_PAD, AGG_W), jnp.float32),
        compiler_params=pltpu.CompilerParams(use_tc_tiling_on_sc=False),
        scratch_types=[
            pltpu.VMEM((CH_W, CHUNK), jnp.int32),
            pltpu.VMEM((CHUNK, AGG_W), jnp.float32),
            pltpu.VMEM((CHUNK, AGG_W), jnp.float32),
            pltpu.VMEM((CHUNK, AGG_W), jnp.float32),
            pltpu.VMEM((CHUNK, AGG_W), jnp.float32),
            pltpu.VMEM_SHARED((N_PAD, AGG_W), jnp.float32),
            pltpu.SemaphoreType.DMA,
            pltpu.SemaphoreType.DMA,
            pltpu.SemaphoreType.DMA,
            pltpu.SemaphoreType.DMA,
            pltpu.SemaphoreType.DMA,
            pltpu.SemaphoreType.DMA,
            pltpu.SemaphoreType.DMA,
            pltpu.SemaphoreType.DMA,
        ],
    )
    def sk(m_hbm, idx_hbm, z_hbm, out_hbm, idx_v, mb0, mb1, mb2, mb3, acc,
           ls0, ls1, ls2, ls3, as0, as1, as2, as3):
        c = lax.axis_index("c")
        s = lax.axis_index("s")
        wid = s * NC + c
        pltpu.sync_copy(idx_hbm.at[pl.ds(wid * CH_W, CH_W), :], idx_v)
        pltpu.sync_copy(z_hbm, acc.at[pl.ds(s * STRIPE, STRIPE), :])
        plsc.subcore_barrier()
        ebase = wid * E_W
        bufs = (mb0, mb1, mb2, mb3)
        lsems = (ls0, ls1, ls2, ls3)
        asems = (as0, as1, as2, as3)
        lds = [None] * 4
        ads = [None] * 4

        def load(t):
            q = t % 4
            lds[q] = pltpu.async_copy(
                m_hbm.at[pl.ds(ebase + t * CHUNK, CHUNK), :], bufs[q],
                lsems[q])

        load(0)
        load(1)
        for j in range(CH_W):
            p = j % 4
            nj = j + 2
            if nj < CH_W:
                q = nj % 4
                if ads[q] is not None:
                    ads[q].wait()
                    ads[q] = None
                load(nj)
            lds[p].wait()
            ads[p] = pltpu.async_copy(
                bufs[p], acc.at[idx_v.at[j]], asems[p], add=True)
        for q in range(4):
            if ads[q] is not None:
                ads[q].wait()
        plsc.subcore_barrier()
        pltpu.sync_copy(
            acc.at[pl.ds(s * STRIPE, STRIPE), :],
            out_hbm.at[c, pl.ds(s * STRIPE, STRIPE), :],
        )

    return sk(m_ext, dst2d, zstripe)


def _epi_body(aggc_ref, xp_ref, xn_ref, mws_ref, mwg_ref, mwv_ref,
              uw0_ref, uw1_ref, sw0_ref, sw1_ref, rs_ref, out_ref):
    f32 = jnp.float32
    agg = aggc_ref[0] + aggc_ref[1]
    agg = agg[:N, :]
    den = jnp.maximum(agg[:, 64:65], 1e-8)
    a = agg[:, :64] / den
    a_s = a[:, :16]
    a_vx = a[:, 16:32]
    a_vy = a[:, 32:48]
    a_vz = a[:, 48:64]

    scal = _silu(jnp.dot(a_s, mws_ref[...], preferred_element_type=f32)
                 * _SCL)
    gts = jax.nn.sigmoid(jnp.dot(a_s, mwg_ref[...],
                                 preferred_element_type=f32) * _SCL)
    mwv = mwv_ref[...]
    vex = gts * (jnp.dot(a_vx, mwv, preferred_element_type=f32) * _SCL)
    vey = gts * (jnp.dot(a_vy, mwv, preferred_element_type=f32) * _SCL)
    vez = gts * (jnp.dot(a_vz, mwv, preferred_element_type=f32) * _SCL)

    xn = xn_ref[...]
    sw0 = sw0_ref[...]
    sw1 = sw1_ref[...]
    uw0 = uw0_ref[...]
    uw1 = uw1_ref[...]
    o_s = (jnp.dot(xn[:, :16], sw0, preferred_element_type=f32)
           + jnp.dot(scal, uw0, preferred_element_type=f32)) * _SCL
    o_vx = (jnp.dot(xn[:, 16:32], sw1, preferred_element_type=f32)
            + jnp.dot(vex, uw1, preferred_element_type=f32)) * _SCL
    o_vy = (jnp.dot(xn[:, 32:48], sw1, preferred_element_type=f32)
            + jnp.dot(vey, uw1, preferred_element_type=f32)) * _SCL
    o_vz = (jnp.dot(xn[:, 48:64], sw1, preferred_element_type=f32)
            + jnp.dot(vez, uw1, preferred_element_type=f32)) * _SCL
    out = jnp.concatenate([o_s, o_vx, o_vy, o_vz], axis=1)
    out_ref[...] = xp_ref[...] + rs_ref[0, 0] * out


def _epilogue(aggc, x_p, xn_p, mws, mwg, mwv, uw0, uw1, sw0, sw1, rs):
    return pl.pallas_call(
        _epi_body,
        out_shape=jax.ShapeDtypeStruct((N, D), jnp.float32),
    )(aggc, x_p, xn_p, mws, mwg, mwv, uw0, uw1, sw0, sw1, rs)


def _planarize(arr64):
    v = arr64[:, 16:].reshape(-1, 16, 3).transpose(0, 2, 1).reshape(-1, 48)
    return jnp.concatenate([arr64[:, :16], v], axis=1)


def _unplanarize(arr64):
    v = arr64[:, 16:].reshape(-1, 3, 16).transpose(0, 2, 1).reshape(-1, 48)
    return jnp.concatenate([arr64[:, :16], v], axis=1)


def kernel(x, edge_src, edge_dst, edge_sh, edge_rbf, edge_len, norm_w,
           norm_b, mlp_w1, mlp_b1, mlp_w2, mlp_b2, mlp_w3, mlp_b3, gate_w1,
           gate_b1, gate_w2, gate_b2, msg_ws, msg_wg, msg_wv, upd_w0, upd_w1,
           self_w0, self_w1, res_scale):
    f32 = jnp.float32
    x_p = _planarize(x)
    wv = norm_w[16:].reshape(16, 3).T.reshape(48)
    w_p = jnp.concatenate([norm_w[:16], wv]).reshape(1, D)
    bv = norm_b[16:].reshape(16, 3).T.reshape(48)
    b_p = jnp.concatenate([norm_b[:16], bv]).reshape(1, D)

    pad = E_PAD - E
    src_p = jnp.pad(edge_src.astype(jnp.int32), (0, pad)).reshape(TOT_CH,
                                                                  CHUNK)
    dst_p = jnp.pad(edge_dst.astype(jnp.int32), (0, pad),
                    constant_values=TRASH).reshape(TOT_CH, CHUNK)
    len2d = jnp.pad(edge_len, (0, E_CUT - E)).reshape(CUT_R, 128)

    zstripe = jnp.zeros((STRIPE, AGG_W), f32)
    R4m = jnp.asarray(_R4_np)
    S4m = jnp.asarray(_S4_np)

    xn_p, cut2d = _prep(x_p, w_p, b_p, len2d)
    cut = cut2d.reshape(E_CUT)[:E].reshape(E, 1)
    xs = _gather_sc(xn_p, src_p)
    m_ext = _main(edge_rbf, edge_sh, cut, xs,
                  mlp_w1, mlp_b1.reshape(1, -1), mlp_w2,
                  mlp_b2.reshape(1, -1), mlp_w3, mlp_b3.reshape(1, -1),
                  gate_w1, gate_b1.reshape(1, -1), gate_w2,
                  gate_b2.reshape(1, -1), R4m, S4m)
    aggc = _scatter_sc(m_ext, dst_p, zstripe)
    out_p = _epilogue(aggc, x_p, xn_p, msg_ws, msg_wg, msg_wv, upd_w0,
                      upd_w1, self_w0, self_w1,
                      res_scale.reshape(1, 1))
    return _unplanarize(out_p)


# speedup vs baseline: 3.3765x; 1.0715x over previous
import functools

import numpy as np
import jax
import jax.numpy as jnp
from jax import lax
from jax.experimental import pallas as pl
from jax.experimental.pallas import tpu as pltpu
from jax.experimental.pallas import tpu_sc as plsc

N = 10000
E = 160000
MUL = 16
D = 64
W128 = 128
CUTOFF = 1.0
EPS = 1e-8

NC = 2
NS = 16
NW = NC * NS
CHUNK = 128
CH_PER_GROUP = 5
GROUPS = 8
CH_W = CH_PER_GROUP * GROUPS
E_W = CH_W * CHUNK
E_PAD = NW * E_W
GROUP_E = CH_PER_GROUP * CHUNK
TOT_CH = NW * CH_W
N_PAD = 10240
TRASH = N_PAD - 1
STRIPE = N_PAD // NS

EB = 2000
N_EB = E // EB

CUT_R = 1256
E_CUT = CUT_R * 128

_C3 = float(1.0 / np.sqrt(3.0))
_P0 = float(1.0 / np.sqrt(2.0 * MUL))
_P1C = float(np.sqrt(3.0 / (2.0 * MUL)) / np.sqrt(3.0))
_SCL = float(1.0 / np.sqrt(MUL))

_P_np = np.zeros((64, 64), np.float32)
for _j in range(16):
    _P_np[_j, _j] = 1.0
for _u in range(16):
    for _k in range(3):
        _P_np[16 + 3 * _u + _k, 16 + 16 * _k + _u] = 1.0

_R4_np = np.zeros((64, 1024), np.float32)
for _g in range(4):
    for _u in range(16):
        _R4_np[_g * 16 + _u, _g * 256 + _u * 16:_g * 256 + (_u + 1) * 16] = 1.0
_S4_np = np.zeros((1024, 64), np.float32)
for _q, _sc in enumerate([_P0, _P1C, _P1C, _P1C]):
    for _u in range(16):
        for _w in range(16):
            _S4_np[_q * 256 + _u * 16 + _w, _q * 16 + _w] = _sc


def _silu(v):
    return v * jax.nn.sigmoid(v)


def _prep_body(x_ref, w_ref, b_ref, p_ref, len_ref, out_ref, cut_ref):
    f32 = jnp.float32
    P = p_ref[...]
    xp = jnp.dot(x_ref[...], P, preferred_element_type=f32)
    w_p = jnp.dot(w_ref[...], P, preferred_element_type=f32)
    b_p = jnp.dot(b_ref[...], P, preferred_element_type=f32)
    s = xp[:, :MUL]
    mean = jnp.mean(s, axis=1, keepdims=True)
    var = jnp.mean((s - mean) ** 2, axis=1, keepdims=True)
    sn = (s - mean) * lax.rsqrt(var + EPS)
    vx = xp[:, 16:32]
    vy = xp[:, 32:48]
    vz = xp[:, 48:64]
    ninv = lax.rsqrt((vx * vx + vy * vy + vz * vz) * (1.0 / 3.0) + EPS)
    out = jnp.concatenate([sn, vx * ninv, vy * ninv, vz * ninv], axis=1)
    out = out * w_p + b_p
    out_ref[...] = jnp.concatenate(
        [out, jnp.zeros((N, 64), f32)], axis=1)
    el = len_ref[...]
    cut_ref[...] = (0.5 * (jnp.cos(jnp.pi * el) + 1.0)
                    * (el <= CUTOFF).astype(f32))


def _prep(x, nw, nb, Pm, len2d):
    return pl.pallas_call(
        _prep_body,
        out_shape=(jax.ShapeDtypeStruct((N, W128), jnp.float32),
                   jax.ShapeDtypeStruct((CUT_R, 128), jnp.float32)),
    )(x, nw, nb, Pm, len2d)


def _gather_sc(table, idx2d):
    mesh = plsc.VectorSubcoreMesh(core_axis_name="c", subcore_axis_name="s")

    @functools.partial(
        pl.kernel,
        mesh=mesh,
        out_type=jax.ShapeDtypeStruct((E_PAD, W128), jnp.float32),
        scratch_types=[
            pltpu.VMEM((CH_W, CHUNK), jnp.int32),
            pltpu.VMEM((GROUP_E, W128), jnp.float32),
            pltpu.SemaphoreType.DMA,
        ],
    )
    def gk(tab_hbm, idx_hbm, out_hbm, idx_v, gbuf, sem):
        c = lax.axis_index("c")
        s = lax.axis_index("s")
        wid = s * NC + c
        pltpu.sync_copy(idx_hbm.at[pl.ds(wid * CH_W, CH_W), :], idx_v)
        ebase = wid * E_W
        for g in range(GROUPS):
            cps = []
            for j in range(CH_PER_GROUP):
                cps.append(pltpu.async_copy(
                    tab_hbm.at[idx_v.at[g * CH_PER_GROUP + j]],
                    gbuf.at[pl.ds(j * CHUNK, CHUNK), :],
                    sem,
                ))
            for cp in cps:
                cp.wait()
            pltpu.sync_copy(
                gbuf, out_hbm.at[pl.ds(ebase + g * GROUP_E, GROUP_E), :])

    return gk(table, idx2d)


def _main_body(rbf_ref, aux_ref, xs_ref, w1_ref, b1_ref, w2_ref,
               b2_ref, w3_ref, b3_ref, gw1_ref, gb1_ref, gw2_ref, gb2_ref,
               r4_ref, s4_ref, out_ref):
    f32 = jnp.float32
    rbf = rbf_ref[...]
    h = _silu(jnp.dot(rbf, w1_ref[...], preferred_element_type=f32)
              + b1_ref[...])
    h = _silu(jnp.dot(h, w2_ref[...], preferred_element_type=f32)
              + b2_ref[...])
    tpw = jnp.dot(h, w3_ref[...], preferred_element_type=f32) + b3_ref[...]

    xs = xs_ref[...]
    aux = aux_ref[...]
    sh0 = aux[:, 0:1]
    shx = aux[:, 1:2]
    shy = aux[:, 2:3]
    shz = aux[:, 3:4]
    cut = aux[:, 4:5]

    XR = jnp.dot(xs[:, :64], r4_ref[...], preferred_element_type=f32)
    sr = XR[:, 0:256]
    vxr = XR[:, 256:512]
    vyr = XR[:, 512:768]
    vzr = XR[:, 768:1024]

    ar = sr * sh0
    br = vxr * shx + vyr * shy + vzr * shz
    t0 = tpw[:, 0:256] * ar + _C3 * (tpw[:, 768:1024] * br)
    c011 = tpw[:, 256:512] * sr
    tq = tpw[:, 512:768]
    txp = c011 * shx + (tq * vxr) * sh0
    typ = c011 * shy + (tq * vyr) * sh0
    tzp = c011 * shz + (tq * vzr) * sh0
    T = jnp.concatenate([t0, txp, typ, tzp], axis=1)
    m = jnp.dot(T, s4_ref[...], preferred_element_type=f32)

    g = _silu(jnp.dot(rbf, gw1_ref[...], preferred_element_type=f32)
              + gb1_ref[...])
    gw = jax.nn.sigmoid(jnp.dot(g, gw2_ref[...], preferred_element_type=f32)
                        + gb2_ref[...])
    ew = cut * gw

    m = m * ew
    ew16 = jnp.broadcast_to(ew, (m.shape[0], 16))
    pad48 = jnp.zeros((m.shape[0], 48), f32)
    out_ref[...] = jnp.concatenate([m, ew16, pad48], axis=1)


def _main(rbf, aux, xs, w1, b1, w2, b2, w3, b3, gw1, gb1, gw2, gb2,
          R4m, S4m):
    blk = lambda shp: pl.BlockSpec(shp, lambda i: (0, 0))
    ebk = lambda w: pl.BlockSpec((EB, w), lambda i: (i, 0))
    return pl.pallas_call(
        _main_body,
        grid=(N_EB,),
        in_specs=[
            ebk(16), ebk(8), ebk(W128),
            blk((16, 64)), blk((1, 64)), blk((64, 64)), blk((1, 64)),
            blk((64, 1024)), blk((1, 1024)),
            blk((16, 64)), blk((1, 64)), blk((64, 1)), blk((1, 1)),
            blk((64, 1024)), blk((1024, 64)),
        ],
        out_specs=pl.BlockSpec((EB, W128), lambda i: (i, 0)),
        out_shape=jax.ShapeDtypeStruct((E_PAD, W128), jnp.float32),
    )(rbf, aux, xs, w1, b1, w2, b2, w3, b3, gw1, gb1, gw2, gb2,
      R4m, S4m)


def _scatter_sc(m_ext, dst2d, zstripe):
    mesh = plsc.VectorSubcoreMesh(core_axis_name="c", subcore_axis_name="s")

    @functools.partial(
        pl.kernel,
        mesh=mesh,
        out_type=jax.ShapeDtypeStruct((NC, N_PAD, W128), jnp.float32),
        scratch_types=[
            pltpu.VMEM((CH_W, CHUNK), jnp.int32),
            pltpu.VMEM((CHUNK, W128), jnp.float32),
            pltpu.VMEM((CHUNK, W128), jnp.float32),
            pltpu.VMEM_SHARED((N_PAD, W128), jnp.float32),
            pltpu.SemaphoreType.DMA,
            pltpu.SemaphoreType.DMA,
            pltpu.SemaphoreType.DMA,
            pltpu.SemaphoreType.DMA,
        ],
    )
    def sk(m_hbm, idx_hbm, z_hbm, out_hbm, idx_v, mb0, mb1, acc,
           ls0, ls1, as0, as1):
        c = lax.axis_index("c")
        s = lax.axis_index("s")
        wid = s * NC + c
        pltpu.sync_copy(idx_hbm.at[pl.ds(wid * CH_W, CH_W), :], idx_v)
        pltpu.sync_copy(z_hbm, acc.at[pl.ds(s * STRIPE, STRIPE), :])
        plsc.subcore_barrier()
        ebase = wid * E_W
        bufs = (mb0, mb1)
        lsems = (ls0, ls1)
        asems = (as0, as1)
        lds = [None, None]
        ads = [None, None]

        def load(t):
            q = t % 2
            lds[q] = pltpu.async_copy(
                m_hbm.at[pl.ds(ebase + t * CHUNK, CHUNK), :], bufs[q],
                lsems[q])

        load(0)
        for j in range(CH_W):
            p = j % 2
            nj = j + 1
            if nj < CH_W:
                q = nj % 2
                if ads[q] is not None:
                    ads[q].wait()
                    ads[q] = None
                load(nj)
            lds[p].wait()
            ads[p] = pltpu.async_copy(
                bufs[p], acc.at[idx_v.at[j]], asems[p], add=True)
        for q in range(2):
            if ads[q] is not None:
                ads[q].wait()
        plsc.subcore_barrier()
        pltpu.sync_copy(
            acc.at[pl.ds(s * STRIPE, STRIPE), :],
            out_hbm.at[c, pl.ds(s * STRIPE, STRIPE), :],
        )

    return sk(m_ext, dst2d, zstripe)


def _epi_body(aggc_ref, x_ref, xn_ref, mws_ref, mwg_ref, mwv_ref,
              uw0_ref, uw1_ref, sw0_ref, sw1_ref, rs_ref, pt_ref, out_ref):
    f32 = jnp.float32
    agg = aggc_ref[0] + aggc_ref[1]
    agg = agg[:N, :]
    den = jnp.maximum(agg[:, 64:65], 1e-8)
    a = agg[:, :64] / den
    a_s = a[:, :16]
    a_vx = a[:, 16:32]
    a_vy = a[:, 32:48]
    a_vz = a[:, 48:64]

    scal = _silu(jnp.dot(a_s, mws_ref[...], preferred_element_type=f32)
                 * _SCL)
    gts = jax.nn.sigmoid(jnp.dot(a_s, mwg_ref[...],
                                 preferred_element_type=f32) * _SCL)
    mwv = mwv_ref[...]
    vex = gts * (jnp.dot(a_vx, mwv, preferred_element_type=f32) * _SCL)
    vey = gts * (jnp.dot(a_vy, mwv, preferred_element_type=f32) * _SCL)
    vez = gts * (jnp.dot(a_vz, mwv, preferred_element_type=f32) * _SCL)

    xn = xn_ref[...]
    sw0 = sw0_ref[...]
    sw1 = sw1_ref[...]
    uw0 = uw0_ref[...]
    uw1 = uw1_ref[...]
    o_s = (jnp.dot(xn[:, :16], sw0, preferred_element_type=f32)
           + jnp.dot(scal, uw0, preferred_element_type=f32)) * _SCL
    o_vx = (jnp.dot(xn[:, 16:32], sw1, preferred_element_type=f32)
            + jnp.dot(vex, uw1, preferred_element_type=f32)) * _SCL
    o_vy = (jnp.dot(xn[:, 32:48], sw1, preferred_element_type=f32)
            + jnp.dot(vey, uw1, preferred_element_type=f32)) * _SCL
    o_vz = (jnp.dot(xn[:, 48:64], sw1, preferred_element_type=f32)
            + jnp.dot(vez, uw1, preferred_element_type=f32)) * _SCL
    out_p = jnp.concatenate([o_s, o_vx, o_vy, o_vz], axis=1)
    out = jnp.dot(out_p, pt_ref[...], preferred_element_type=f32)
    out_ref[...] = x_ref[...] + rs_ref[0, 0] * out


def _epilogue(aggc, x, xn_p, mws, mwg, mwv, uw0, uw1, sw0, sw1, rs, PTm):
    return pl.pallas_call(
        _epi_body,
        out_shape=jax.ShapeDtypeStruct((N, D), jnp.float32),
    )(aggc, x, xn_p, mws, mwg, mwv, uw0, uw1, sw0, sw1, rs, PTm)


def kernel(x, edge_src, edge_dst, edge_sh, edge_rbf, edge_len, norm_w,
           norm_b, mlp_w1, mlp_b1, mlp_w2, mlp_b2, mlp_w3, mlp_b3, gate_w1,
           gate_b1, gate_w2, gate_b2, msg_ws, msg_wg, msg_wv, upd_w0, upd_w1,
           self_w0, self_w1, res_scale):
    f32 = jnp.float32
    pad = E_PAD - E
    src_p = jnp.pad(edge_src.astype(jnp.int32), (0, pad)).reshape(TOT_CH,
                                                                  CHUNK)
    dst_p = jnp.pad(edge_dst.astype(jnp.int32), (0, pad),
                    constant_values=TRASH).reshape(TOT_CH, CHUNK)
    len2d = jnp.pad(edge_len, (0, E_CUT - E)).reshape(CUT_R, 128)

    zstripe = jnp.zeros((STRIPE, W128), f32)
    Pm = jnp.asarray(_P_np)
    PTm = jnp.asarray(_P_np.T)
    R4m = jnp.asarray(_R4_np)
    S4m = jnp.asarray(_S4_np)

    xn_p, cut2d = _prep(x, norm_w.reshape(1, D), norm_b.reshape(1, D), Pm,
                        len2d)
    cut = cut2d.reshape(E_CUT)[:E].reshape(E, 1)
    aux = jnp.concatenate([edge_sh, cut, jnp.zeros((E, 3), f32)], axis=1)
    xs = _gather_sc(xn_p, src_p)
    m_ext = _main(edge_rbf, aux, xs,
                  mlp_w1, mlp_b1.reshape(1, -1), mlp_w2,
                  mlp_b2.reshape(1, -1), mlp_w3, mlp_b3.reshape(1, -1),
                  gate_w1, gate_b1.reshape(1, -1), gate_w2,
                  gate_b2.reshape(1, -1), R4m, S4m)
    aggc = _scatter_sc(m_ext, dst_p, zstripe)
    return _epilogue(aggc, x, xn_p, msg_ws, msg_wg, msg_wv, upd_w0,
                     upd_w1, self_w0, self_w1,
                     res_scale.reshape(1, 1), PTm)
